# baseline (speedup 1.0000x reference)
import functools

import jax
import jax.numpy as jnp
from jax import lax
from jax.experimental import pallas as pl
from jax.experimental.pallas import tpu as pltpu
from jax.experimental.pallas import tpu_sc as plsc

_NC = 2
_NS = 16
_R = 4
_NBUF = 8


def _start(src, dst, sem):
    pltpu.async_copy(src, dst, sem)


def _add_chunk(xb, pb, rows, d):
    for r in range(rows):
        @plsc.parallel_loop(0, d, 16, unroll=16)
        def _(col):
            plsc.addupdate(xb.at[r, pl.ds(col, 16)], pb[r, pl.ds(col, 16)])


def kernel(x, pe_weight):
    B, S, D = x.shape
    nw = _NC * _NS
    seq_per_w = S // nw
    n_seq_chunks = seq_per_w // _R
    n_groups = n_seq_chunks // 2

    mesh = plsc.VectorSubcoreMesh(core_axis_name="c", subcore_axis_name="s")

    vmem_chunk = pltpu.VMEM((_R, D), jnp.float32)
    @functools.partial(
        pl.kernel,
        mesh=mesh,
        out_type=jax.ShapeDtypeStruct((B, S, D), x.dtype),
        scratch_types=(
            [vmem_chunk] * _NBUF + [vmem_chunk] * 2
            + [pltpu.SemaphoreType.DMA] * (2 * _NBUF + 2)
        ),
    )
    def run(x_hbm, pe_hbm, out_hbm, *scr):
        xbs = scr[0:_NBUF]
        pbs = scr[_NBUF:_NBUF + 2]
        lsem = scr[_NBUF + 2:2 * _NBUF + 2]
        ssem = scr[2 * _NBUF + 2:3 * _NBUF + 2]
        psem = scr[3 * _NBUF + 2:3 * _NBUF + 4]

        wid = lax.axis_index("s") * _NC + lax.axis_index("c")
        sbase = wid * seq_per_w

        def x_slice(j, b):
            return x_hbm.at[b, pl.ds(sbase + j * _R, _R)]

        def out_slice(j, b):
            return out_hbm.at[b, pl.ds(sbase + j * _R, _R)]

        def pe_slice(j):
            return pe_hbm.at[pl.ds(sbase + j * _R, _R)]

        pltpu.async_copy(pe_slice(0), pbs[0], psem[0])
        for b in range(4):
            pltpu.async_copy(x_slice(0, b), xbs[b], lsem[b])

        def group(g, carry):
            for i in range(8):
                b = i % 4
                j = 2 * g + (1 if i >= 4 else 0)
                p = (i + 4) % 8
                if i == 0:
                    pltpu.make_async_copy(pe_slice(2 * g), pbs[0], psem[0]).wait()
                    pltpu.async_copy(pe_slice(2 * g + 1), pbs[1], psem[1])
                if i == 4:
                    pltpu.make_async_copy(pe_slice(2 * g + 1), pbs[1], psem[1]).wait()
                    pl.when(g < n_groups - 1)(
                        lambda: _start(pe_slice(2 * g + 2), pbs[0], psem[0]))

                jn = 2 * g + 1 + (1 if i >= 4 else 0)
                jo = 2 * g - (1 if i < 4 else 0)
                store_wait = lambda: pltpu.make_async_copy(
                    xbs[p], out_slice(jo, b), ssem[p]).wait()
                load_next = lambda: _start(x_slice(jn, b), xbs[p], lsem[p])
                if i < 4:
                    pl.when(g > 0)(store_wait)
                    load_next()
                else:
                    store_wait()
                    pl.when(g < n_groups - 1)(load_next)

                pltpu.make_async_copy(x_slice(j, b), xbs[i], lsem[i]).wait()
                _add_chunk(xbs[i], pbs[0 if i < 4 else 1], _R, D)
                pltpu.async_copy(xbs[i], out_slice(j, b), ssem[i])
            return carry

        lax.fori_loop(0, n_groups, group, 0)

        for i in range(4, 8):
            pltpu.make_async_copy(
                xbs[i], out_slice(2 * n_groups - 1, i % 4), ssem[i]).wait()

    return run(x, pe_weight)


# speedup vs baseline: 1.0270x; 1.0270x over previous
import functools

import jax
import jax.numpy as jnp
from jax import lax
from jax.experimental import pallas as pl
from jax.experimental.pallas import tpu as pltpu
from jax.experimental.pallas import tpu_sc as plsc

_NC = 2
_NS = 16
_R = 4
_NBUF = 8


def _start(src, dst, sem):
    pltpu.async_copy(src, dst, sem)


def _add_chunk(xb, pb, rows, d):
    for r in range(rows):
        @plsc.parallel_loop(0, d, 16, unroll=8)
        def _(col):
            plsc.addupdate(xb.at[r, pl.ds(col, 16)], pb[r, pl.ds(col, 16)])


def kernel(x, pe_weight):
    B, S, D = x.shape
    nw = _NC * _NS
    seq_per_w = S // nw
    n_seq_chunks = seq_per_w // _R
    n_groups = n_seq_chunks // 2

    mesh = plsc.VectorSubcoreMesh(core_axis_name="c", subcore_axis_name="s")

    vmem_chunk = pltpu.VMEM((_R, D), jnp.float32)
    @functools.partial(
        pl.kernel,
        mesh=mesh,
        out_type=jax.ShapeDtypeStruct((B, S, D), x.dtype),
        scratch_types=(
            [vmem_chunk] * _NBUF + [vmem_chunk] * 2
            + [pltpu.SemaphoreType.DMA] * (2 * _NBUF + 2)
        ),
    )
    def run(x_hbm, pe_hbm, out_hbm, *scr):
        xbs = scr[0:_NBUF]
        pbs = scr[_NBUF:_NBUF + 2]
        lsem = scr[_NBUF + 2:2 * _NBUF + 2]
        ssem = scr[2 * _NBUF + 2:3 * _NBUF + 2]
        psem = scr[3 * _NBUF + 2:3 * _NBUF + 4]

        wid = lax.axis_index("s") * _NC + lax.axis_index("c")
        sbase = wid * seq_per_w

        def x_slice(j, b):
            return x_hbm.at[b, pl.ds(sbase + j * _R, _R)]

        def out_slice(j, b):
            return out_hbm.at[b, pl.ds(sbase + j * _R, _R)]

        def pe_slice(j):
            return pe_hbm.at[pl.ds(sbase + j * _R, _R)]

        pltpu.async_copy(pe_slice(0), pbs[0], psem[0])
        for b in range(4):
            pltpu.async_copy(x_slice(0, b), xbs[b], lsem[b])

        def group(g, carry):
            for i in range(8):
                b = i % 4
                j = 2 * g + (1 if i >= 4 else 0)
                p = (i + 4) % 8
                if i == 0:
                    pltpu.make_async_copy(pe_slice(2 * g), pbs[0], psem[0]).wait()
                    pltpu.async_copy(pe_slice(2 * g + 1), pbs[1], psem[1])
                if i == 4:
                    pltpu.make_async_copy(pe_slice(2 * g + 1), pbs[1], psem[1]).wait()
                    pl.when(g < n_groups - 1)(
                        lambda: _start(pe_slice(2 * g + 2), pbs[0], psem[0]))

                jn = 2 * g + 1 + (1 if i >= 4 else 0)
                jo = 2 * g - (1 if i < 4 else 0)
                store_wait = lambda: pltpu.make_async_copy(
                    xbs[p], out_slice(jo, b), ssem[p]).wait()
                load_next = lambda: _start(x_slice(jn, b), xbs[p], lsem[p])
                if i < 4:
                    pl.when(g > 0)(store_wait)
                    load_next()
                else:
                    store_wait()
                    pl.when(g < n_groups - 1)(load_next)

                pltpu.make_async_copy(x_slice(j, b), xbs[i], lsem[i]).wait()
                _add_chunk(xbs[i], pbs[0 if i < 4 else 1], _R, D)
                pltpu.async_copy(xbs[i], out_slice(j, b), ssem[i])
            return carry

        lax.fori_loop(0, n_groups, group, 0)

        for i in range(4, 8):
            pltpu.make_async_copy(
                xbs[i], out_slice(2 * n_groups - 1, i % 4), ssem[i]).wait()

    return run(x, pe_weight)
